# baseline (speedup 1.0000x reference)
import functools

import jax
import jax.numpy as jnp
from jax import lax
from jax.experimental import pallas as pl
from jax.experimental.pallas import tpu as pltpu
from jax.experimental.pallas import tpu_sc as plsc

Q = 4096
K = 100000
D = 512
N_CLASSES = 64
KT = 2000


QS = 1024


def _knn_body(support_ref, labels_ref, query_ref, w_ref,
              idx_out_ref, packed_out_ref,
              qnt_ref, scores_ref, bestv_ref, bestk_ref):
    i = pl.program_id(0)
    nk = pl.num_programs(0)

    @pl.when(i == 0)
    def _init_queries():
        for s in range(Q // QS):
            rowsl = pl.ds(s * QS, QS)
            qe = jnp.dot(query_ref[rowsl, :], w_ref[...],
                         preferred_element_type=jnp.float32)
            sq = jnp.sum(qe * qe, axis=1, keepdims=True)
            qn = qe * lax.rsqrt(jnp.maximum(sq, 1e-12))
            qnt_ref[:, pl.ds(s * QS, QS)] = qn.T
        bestv_ref[...] = jnp.full((8, Q), -jnp.inf, jnp.float32)
        bestk_ref[...] = jnp.zeros((8, Q), jnp.int32)

    s_emb = jnp.dot(support_ref[...], w_ref[...],
                    preferred_element_type=jnp.float32)
    sq = jnp.sum(s_emb * s_emb, axis=1, keepdims=True)
    s_n = s_emb * lax.rsqrt(jnp.maximum(sq, 1e-12))
    scores_ref[...] = jnp.dot(s_n, qnt_ref[...],
                              preferred_element_type=jnp.float32)

    labs3 = labels_ref[...].reshape(KT // 2, 2, N_CLASSES)
    packed_out_ref[...] = jnp.concatenate(
        [labs3[:, 0, :], labs3[:, 1, :]], axis=1)

    base = i * KT
    for s in range(Q // QS):
        colsl = pl.ds(s * QS, QS)
        bv = bestv_ref[:, colsl]
        bk = bestk_ref[:, colsl]
        for c in range(KT // 8):
            chunk = scores_ref[pl.ds(c * 8, 8), colsl]
            upd = chunk > bv
            bv = jnp.where(upd, chunk, bv)
            bk = jnp.where(upd, base + c * 8, bk)
        bestv_ref[:, colsl] = bv
        bestk_ref[:, colsl] = bk

    @pl.when(i == nk - 1)
    def _finish():
        bv = bestv_ref[...]
        kfull = bestk_ref[...] + lax.broadcasted_iota(jnp.int32, (8, Q), 0)
        m = jnp.max(bv, axis=0, keepdims=True)
        cand = jnp.where(bv == m, kfull, K)
        idx = jnp.min(cand, axis=0, keepdims=True)
        idx_out_ref[...] = jnp.broadcast_to(idx, (8, Q))


def _label_body(rows_ref, idx_ref, out_ref):
    rows128 = rows_ref[...]
    parity = (idx_ref[...] & 1) == 0
    rows = jnp.where(parity, rows128[:, :N_CLASSES], rows128[:, N_CLASSES:])
    rmax = jnp.max(rows, axis=1, keepdims=True)
    cidx = lax.broadcasted_iota(jnp.int32, rows.shape, 1)
    cand = jnp.where(rows == rmax, cidx, N_CLASSES)
    amax = jnp.min(cand, axis=1, keepdims=True)
    out_ref[...] = (cidx == amax).astype(jnp.float32)


def _make_sc_gather():
    info = plsc.get_sparse_core_info()
    nw = info.num_cores * info.num_subcores
    b_per_w = Q // nw
    mesh = plsc.VectorSubcoreMesh(core_axis_name="c", subcore_axis_name="s")

    @functools.partial(
        pl.kernel, mesh=mesh,
        out_type=jax.ShapeDtypeStruct((Q, 2 * N_CLASSES), jnp.float32),
        scratch_types=[
            pltpu.VMEM((b_per_w,), jnp.int32),
            pltpu.VMEM((b_per_w,), jnp.int32),
            pltpu.VMEM((b_per_w, 2 * N_CLASSES), jnp.float32),
            pltpu.SemaphoreType.DMA,
        ],
    )
    def gather(table_hbm, idx_hbm, out_hbm, idx_v, idx_half, rows_v, sem):
        wid = lax.axis_index("s") * info.num_cores + lax.axis_index("c")
        base = wid * b_per_w
        pltpu.sync_copy(idx_hbm.at[pl.ds(base, b_per_w)], idx_v)
        for j in range(b_per_w // 16):
            sl = pl.ds(j * 16, 16)
            idx_half[sl] = idx_v[sl] >> 1
        pltpu.async_copy(table_hbm.at[idx_half], rows_v, sem).wait()
        pltpu.sync_copy(rows_v, out_hbm.at[pl.ds(base, b_per_w)])

    return gather


_sc_gather = None


def kernel(support_examples, query_examples, support_labels_onehot, W):
    global _sc_gather
    if _sc_gather is None:
        _sc_gather = _make_sc_gather()

    nk = K // KT
    idx_rows, packed_table = pl.pallas_call(
        _knn_body,
        grid=(nk,),
        in_specs=[
            pl.BlockSpec((KT, D), lambda i: (i, 0)),
            pl.BlockSpec((KT, N_CLASSES), lambda i: (i, 0)),
            pl.BlockSpec((Q, D), lambda i: (0, 0)),
            pl.BlockSpec((D, D), lambda i: (0, 0)),
        ],
        out_specs=[
            pl.BlockSpec((8, Q), lambda i: (0, 0)),
            pl.BlockSpec((KT // 2, 2 * N_CLASSES), lambda i: (i, 0)),
        ],
        out_shape=[
            jax.ShapeDtypeStruct((8, Q), jnp.int32),
            jax.ShapeDtypeStruct((K // 2, 2 * N_CLASSES), jnp.float32),
        ],
        scratch_shapes=[
            pltpu.VMEM((D, Q), jnp.float32),
            pltpu.VMEM((KT, Q), jnp.float32),
            pltpu.VMEM((8, Q), jnp.float32),
            pltpu.VMEM((8, Q), jnp.int32),
        ],
    )(support_examples, support_labels_onehot, query_examples, W)
    best_idx = idx_rows[0].reshape(Q, 1)

    rows = _sc_gather(packed_table, best_idx.reshape(Q))

    labels_prob = pl.pallas_call(
        _label_body,
        out_shape=jax.ShapeDtypeStruct((Q, N_CLASSES), jnp.float32),
    )(rows, best_idx)
    return labels_prob


# speedup vs baseline: 1.0512x; 1.0512x over previous
import functools

import jax
import jax.numpy as jnp
from jax import lax
from jax.experimental import pallas as pl
from jax.experimental.pallas import tpu as pltpu
from jax.experimental.pallas import tpu_sc as plsc

Q = 4096
K = 100000
D = 512
N_CLASSES = 64
KT = 2000


QS = 1024


def _knn_body(support_ref, query_ref, w_ref, idx_out_ref,
              qnt_ref, scores_ref, bestv_ref, bestk_ref):
    i = pl.program_id(0)
    nk = pl.num_programs(0)

    @pl.when(i == 0)
    def _init_queries():
        for s in range(Q // QS):
            rowsl = pl.ds(s * QS, QS)
            qe = jnp.dot(query_ref[rowsl, :], w_ref[...],
                         preferred_element_type=jnp.float32)
            sq = jnp.sum(qe * qe, axis=1, keepdims=True)
            qn = qe * lax.rsqrt(jnp.maximum(sq, 1e-12))
            qnt_ref[:, pl.ds(s * QS, QS)] = qn.T
        bestv_ref[...] = jnp.full((8, Q), -jnp.inf, jnp.float32)
        bestk_ref[...] = jnp.zeros((8, Q), jnp.int32)

    s_emb = jnp.dot(support_ref[...], w_ref[...],
                    preferred_element_type=jnp.float32)
    sq = jnp.sum(s_emb * s_emb, axis=1, keepdims=True)
    s_n = s_emb * lax.rsqrt(jnp.maximum(sq, 1e-12))
    scores_ref[...] = jnp.dot(s_n, qnt_ref[...],
                              preferred_element_type=jnp.float32)

    base = i * KT
    for s in range(Q // QS):
        colsl = pl.ds(s * QS, QS)
        bv = bestv_ref[:, colsl]
        bk = bestk_ref[:, colsl]
        for c in range(KT // 8):
            chunk = scores_ref[pl.ds(c * 8, 8), colsl]
            upd = chunk > bv
            bv = jnp.where(upd, chunk, bv)
            bk = jnp.where(upd, base + c * 8, bk)
        bestv_ref[:, colsl] = bv
        bestk_ref[:, colsl] = bk

    @pl.when(i == nk - 1)
    def _finish():
        bv = bestv_ref[...]
        kfull = bestk_ref[...] + lax.broadcasted_iota(jnp.int32, (8, Q), 0)
        m = jnp.max(bv, axis=0, keepdims=True)
        cand = jnp.where(bv == m, kfull, K)
        idx = jnp.min(cand, axis=0, keepdims=True)
        idx_out_ref[...] = jnp.broadcast_to(idx, (8, Q))


def _label_body(rows_ref, idx_ref, out_ref):
    rows128 = rows_ref[...]
    parity = (idx_ref[...] & 1) == 0
    rows = jnp.where(parity, rows128[:, :N_CLASSES], rows128[:, N_CLASSES:])
    rmax = jnp.max(rows, axis=1, keepdims=True)
    cidx = lax.broadcasted_iota(jnp.int32, rows.shape, 1)
    cand = jnp.where(rows == rmax, cidx, N_CLASSES)
    amax = jnp.min(cand, axis=1, keepdims=True)
    out_ref[...] = (cidx == amax).astype(jnp.float32)


def _make_sc_gather():
    info = plsc.get_sparse_core_info()
    nw = info.num_cores * info.num_subcores
    b_per_w = Q // nw
    mesh = plsc.VectorSubcoreMesh(core_axis_name="c", subcore_axis_name="s")

    @functools.partial(
        pl.kernel, mesh=mesh,
        out_type=jax.ShapeDtypeStruct((Q, 2 * N_CLASSES), jnp.float32),
        scratch_types=[
            pltpu.VMEM((b_per_w,), jnp.int32),
            pltpu.VMEM((b_per_w,), jnp.int32),
            pltpu.VMEM((b_per_w, 2 * N_CLASSES), jnp.float32),
            pltpu.SemaphoreType.DMA,
        ],
    )
    def gather(table_hbm, idx_hbm, out_hbm, idx_v, idx_half, rows_v, sem):
        wid = lax.axis_index("s") * info.num_cores + lax.axis_index("c")
        base = wid * b_per_w
        pltpu.sync_copy(idx_hbm.at[pl.ds(base, b_per_w)], idx_v)
        for j in range(b_per_w // 16):
            sl = pl.ds(j * 16, 16)
            idx_half[sl] = idx_v[sl] >> 1
        pltpu.async_copy(table_hbm.at[idx_half], rows_v, sem).wait()
        pltpu.sync_copy(rows_v, out_hbm.at[pl.ds(base, b_per_w)])

    return gather


_sc_gather = None


def kernel(support_examples, query_examples, support_labels_onehot, W):
    global _sc_gather
    if _sc_gather is None:
        _sc_gather = _make_sc_gather()

    nk = K // KT
    idx_rows = pl.pallas_call(
        _knn_body,
        grid=(nk,),
        in_specs=[
            pl.BlockSpec((KT, D), lambda i: (i, 0)),
            pl.BlockSpec((Q, D), lambda i: (0, 0)),
            pl.BlockSpec((D, D), lambda i: (0, 0)),
        ],
        out_specs=pl.BlockSpec((8, Q), lambda i: (0, 0)),
        out_shape=jax.ShapeDtypeStruct((8, Q), jnp.int32),
        scratch_shapes=[
            pltpu.VMEM((D, Q), jnp.float32),
            pltpu.VMEM((KT, Q), jnp.float32),
            pltpu.VMEM((8, Q), jnp.float32),
            pltpu.VMEM((8, Q), jnp.int32),
        ],
    )(support_examples, query_examples, W)
    best_idx = idx_rows[0].reshape(Q, 1)

    packed_table = support_labels_onehot.reshape(K // 2, 2 * N_CLASSES)
    rows = _sc_gather(packed_table, best_idx.reshape(Q))

    labels_prob = pl.pallas_call(
        _label_body,
        out_shape=jax.ShapeDtypeStruct((Q, N_CLASSES), jnp.float32),
    )(rows, best_idx)
    return labels_prob
